# baseline (speedup 1.0000x reference)
import functools

import jax
import jax.numpy as jnp
from jax import lax
from jax.experimental import pallas as pl
from jax.experimental.pallas import tpu as pltpu
from jax.experimental.pallas import tpu_sc as plsc

_VOCAB = 1000001
_HID = 64
_LABELS = 16
_BATCH = 4096
_SEQ = 200
_CHUNK = 100
_NCHUNK = _SEQ // _CHUNK
_NBUF = 2


def _pool_kernel(ids_hbm, emb_hbm, out_hbm, idx_v, rows_v, out_v, *sems):
    info = plsc.get_sparse_core_info()
    nw = info.num_cores * info.num_subcores
    rows_per_w = _BATCH // nw
    wid = lax.axis_index("s") * info.num_cores + lax.axis_index("c")
    base = wid * rows_per_w

    pltpu.sync_copy(ids_hbm.at[pl.ds(base, rows_per_w)], idx_v)

    def issue(row, buf):
        for c in range(_NCHUNK):
            pltpu.async_copy(
                emb_hbm.at[idx_v.at[row, c]], rows_v.at[buf, c], sems[buf])

    def wait(row, buf):
        for c in range(_NCHUNK):
            pltpu.make_async_copy(
                emb_hbm.at[idx_v.at[row, c]], rows_v.at[buf, c],
                sems[buf]).wait()

    for b in range(_NBUF):
        issue(b, b)

    inv = jnp.full((16,), 1.0 / _SEQ, dtype=jnp.float32)

    def outer(i, _):
        for b in range(_NBUF):
            row = i * _NBUF + b
            wait(row, b)

            def accum(c):
                def body(rr, accs):
                    return tuple(
                        accs[q] + rows_v[b, c, rr, pl.ds(q * 16, 16)]
                        for q in range(4))
                return body

            accs = tuple(jnp.zeros((16,), jnp.float32) for _ in range(4))
            for c in range(_NCHUNK):
                accs = lax.fori_loop(0, _CHUNK, accum(c), accs)

            @pl.when(row + _NBUF < rows_per_w)
            def _():
                issue(row + _NBUF, b)

            for q in range(4):
                out_v[row, pl.ds(q * 16, 16)] = accs[q] * inv
        return 0

    lax.fori_loop(0, rows_per_w // _NBUF, outer, 0)
    pltpu.sync_copy(out_v, out_hbm.at[pl.ds(base, rows_per_w)])


def _make_pool():
    info = plsc.get_sparse_core_info()
    nw = info.num_cores * info.num_subcores
    rows_per_w = _BATCH // nw
    mesh = plsc.VectorSubcoreMesh(core_axis_name="c", subcore_axis_name="s")
    return pl.kernel(
        _pool_kernel,
        out_type=jax.ShapeDtypeStruct((_BATCH, _HID), jnp.float32),
        mesh=mesh,
        scratch_types=[
            pltpu.VMEM((rows_per_w, _NCHUNK, _CHUNK), jnp.int32),
            pltpu.VMEM((_NBUF, _NCHUNK, _CHUNK, _HID), jnp.float32),
            pltpu.VMEM((rows_per_w, _HID), jnp.float32),
        ] + [pltpu.SemaphoreType.DMA] * _NBUF,
        compiler_params=pltpu.CompilerParams(use_tc_tiling_on_sc=False),
    )


def _linear_kernel(x_ref, w_ref, b_ref, o_ref):
    o_ref[...] = lax.dot_general(
        x_ref[...], w_ref[...], (((1,), (1,)), ((), ())),
        preferred_element_type=jnp.float32) + b_ref[...]


def kernel(input_ids, emb, W, b):
    ids = input_ids.astype(jnp.int32).reshape(_BATCH, _NCHUNK, _CHUNK)
    pooled = _make_pool()(ids, emb)
    return pl.pallas_call(
        _linear_kernel,
        out_shape=jax.ShapeDtypeStruct((_BATCH, _LABELS), jnp.float32),
    )(pooled, W, b.reshape(1, _LABELS))


# speedup vs baseline: 1.0724x; 1.0724x over previous
import functools

import jax
import jax.numpy as jnp
from jax import lax
from jax.experimental import pallas as pl
from jax.experimental.pallas import tpu as pltpu
from jax.experimental.pallas import tpu_sc as plsc

_VOCAB = 1000001
_HID = 64
_LABELS = 16
_BATCH = 4096
_SEQ = 200
_CHUNK = 100
_NCHUNK = _SEQ // _CHUNK
_NBUF = 4
_UNROLL = 4


def _pool_kernel(ids_hbm, emb_hbm, out_hbm, idx_v, rows_v, out_v, *sems):
    info = plsc.get_sparse_core_info()
    nw = info.num_cores * info.num_subcores
    rows_per_w = _BATCH // nw
    wid = lax.axis_index("s") * info.num_cores + lax.axis_index("c")
    base = wid * rows_per_w

    pltpu.sync_copy(ids_hbm.at[pl.ds(base, rows_per_w)], idx_v)

    def issue(row, buf):
        for c in range(_NCHUNK):
            pltpu.async_copy(
                emb_hbm.at[idx_v.at[row, c]], rows_v.at[buf, c], sems[buf])

    def wait(row, buf):
        for c in range(_NCHUNK):
            pltpu.make_async_copy(
                emb_hbm.at[idx_v.at[row, c]], rows_v.at[buf, c],
                sems[buf]).wait()

    for b in range(_NBUF):
        issue(b, b)

    inv = jnp.full((16,), 1.0 / _SEQ, dtype=jnp.float32)

    def outer(i, _):
        for b in range(_NBUF):
            row = i * _NBUF + b
            wait(row, b)

            def accum(c):
                def body(it, accs):
                    rr = it * _UNROLL
                    for u in range(_UNROLL):
                        accs = tuple(
                            accs[q] + rows_v[b, c, rr + u, pl.ds(q * 16, 16)]
                            for q in range(4))
                    return accs
                return body

            accs = tuple(jnp.zeros((16,), jnp.float32) for _ in range(4))
            for c in range(_NCHUNK):
                accs = lax.fori_loop(0, _CHUNK // _UNROLL, accum(c), accs)

            @pl.when(row + _NBUF < rows_per_w)
            def _():
                issue(row + _NBUF, b)

            for q in range(4):
                out_v[row, pl.ds(q * 16, 16)] = accs[q] * inv
        return 0

    lax.fori_loop(0, rows_per_w // _NBUF, outer, 0)
    pltpu.sync_copy(out_v, out_hbm.at[pl.ds(base, rows_per_w)])


def _make_pool():
    info = plsc.get_sparse_core_info()
    nw = info.num_cores * info.num_subcores
    rows_per_w = _BATCH // nw
    mesh = plsc.VectorSubcoreMesh(core_axis_name="c", subcore_axis_name="s")
    return pl.kernel(
        _pool_kernel,
        out_type=jax.ShapeDtypeStruct((_BATCH, _HID), jnp.float32),
        mesh=mesh,
        scratch_types=[
            pltpu.VMEM((rows_per_w, _NCHUNK, _CHUNK), jnp.int32),
            pltpu.VMEM((_NBUF, _NCHUNK, _CHUNK, _HID), jnp.float32),
            pltpu.VMEM((rows_per_w, _HID), jnp.float32),
        ] + [pltpu.SemaphoreType.DMA] * _NBUF,
        compiler_params=pltpu.CompilerParams(use_tc_tiling_on_sc=False),
    )


def _linear_kernel(x_ref, w_ref, b_ref, o_ref):
    o_ref[...] = lax.dot_general(
        x_ref[...], w_ref[...], (((1,), (1,)), ((), ())),
        preferred_element_type=jnp.float32) + b_ref[...]


def kernel(input_ids, emb, W, b):
    ids = input_ids.astype(jnp.int32).reshape(_BATCH, _NCHUNK, _CHUNK)
    pooled = _make_pool()(ids, emb)
    return pl.pallas_call(
        _linear_kernel,
        out_shape=jax.ShapeDtypeStruct((_BATCH, _LABELS), jnp.float32),
    )(pooled, W, b.reshape(1, _LABELS))
